# Initial kernel scaffold; baseline (speedup 1.0000x reference)
import jax
import jax.numpy as jnp
from jax.experimental import pallas as pl


def kernel(x, table):
    raise NotImplementedError("write your pallas kernel here")


# speedup vs baseline: 9.6857x; 9.6857x over previous
import functools
import math

import jax
import jax.numpy as jnp
from jax import lax
from jax.experimental import pallas as pl
from jax.experimental.pallas import tpu as pltpu
from jax.experimental.pallas import tpu_sc as plsc


def _sc_gather(x, table):
    (b,) = x.shape
    _, d = table.shape
    info = plsc.get_sparse_core_info()
    nc, ns = info.num_cores, info.num_subcores
    nw = nc * ns
    b_per_w = b // nw
    k = 16
    n_chunks = b_per_w // k

    mesh = plsc.VectorSubcoreMesh(core_axis_name="c", subcore_axis_name="s")

    @functools.partial(
        pl.kernel,
        mesh=mesh,
        out_type=jax.ShapeDtypeStruct((b, d), jnp.float32),
        scratch_types=[
            pltpu.VMEM((b_per_w,), jnp.int32),
            pltpu.VMEM((k, d), jnp.float32),
            pltpu.VMEM((k, d), jnp.float32),
            pltpu.SemaphoreType.DMA,
            pltpu.SemaphoreType.DMA,
        ],
    )
    def gather_kernel(idx_hbm, table_hbm, out_hbm, idx_v, buf0, buf1, sem0, sem1):
        wid = lax.axis_index("s") * nc + lax.axis_index("c")
        base = wid * b_per_w
        pltpu.sync_copy(idx_hbm.at[pl.ds(base, b_per_w)], idx_v)
        bufs = (buf0, buf1)
        sems = (sem0, sem1)
        copies = [None, None]
        copies[0] = pltpu.async_copy(
            table_hbm.at[idx_v.at[pl.ds(0, k)]], bufs[0], sems[0]
        )
        for c in range(n_chunks):
            cur = c % 2
            nxt = (c + 1) % 2
            if c + 1 < n_chunks:
                copies[nxt] = pltpu.async_copy(
                    table_hbm.at[idx_v.at[pl.ds((c + 1) * k, k)]], bufs[nxt], sems[nxt]
                )
            copies[cur].wait()
            pltpu.sync_copy(bufs[cur], out_hbm.at[pl.ds(base + c * k, k)])

    return gather_kernel(x, table)


def _tc_combine(gathered):
    b, d = gathered.shape
    rb = 256
    grid = b // rb
    scale = math.sqrt(float(d))
    neg_log = -math.log(10000.0) / float(d)

    def body(g_ref, o_ref, p_ref, q_ref):
        blk = pl.program_id(0)
        col = lax.broadcasted_iota(jnp.int32, (1, d), 1)
        w = jnp.exp((col - (col % 2)).astype(jnp.float32) * neg_log)

        @pl.when(blk == 0)
        def _():
            r = lax.broadcasted_iota(jnp.int32, (rb, 1), 0).astype(jnp.float32)
            ang = r * w
            p_ref[...] = jnp.cos(ang)
            q_ref[...] = jnp.sin(ang)

        base_ang = (blk * rb).astype(jnp.float32) * w
        sb = jnp.sin(base_ang)
        cb = jnp.cos(base_ang)
        even = (col % 2) == 0
        a = jnp.where(even, sb, cb)
        bv = jnp.where(even, cb, -sb)
        o_ref[...] = g_ref[...] * scale + a * p_ref[...] + bv * q_ref[...]

    return pl.pallas_call(
        body,
        grid=(grid,),
        in_specs=[pl.BlockSpec((rb, d), lambda i: (i, 0))],
        out_specs=pl.BlockSpec((rb, d), lambda i: (i, 0)),
        out_shape=jax.ShapeDtypeStruct((b, d), jnp.float32),
        scratch_shapes=[
            pltpu.VMEM((rb, d), jnp.float32),
            pltpu.VMEM((rb, d), jnp.float32),
        ],
    )(gathered)


def kernel(x, table):
    x = x.astype(jnp.int32)
    gathered = _sc_gather(x, table)
    return _tc_combine(gathered)


# speedup vs baseline: 9.7543x; 1.0071x over previous
import functools
import math

import jax
import jax.numpy as jnp
from jax import lax
from jax.experimental import pallas as pl
from jax.experimental.pallas import tpu as pltpu
from jax.experimental.pallas import tpu_sc as plsc


def _sc_gather(x, table):
    (b,) = x.shape
    _, d = table.shape
    info = plsc.get_sparse_core_info()
    nc, ns = info.num_cores, info.num_subcores
    nw = nc * ns
    b_per_w = b // nw
    k = 16
    n_chunks = b_per_w // k
    nbuf = 3

    mesh = plsc.VectorSubcoreMesh(core_axis_name="c", subcore_axis_name="s")

    @functools.partial(
        pl.kernel,
        mesh=mesh,
        out_type=jax.ShapeDtypeStruct((b, d), jnp.float32),
        scratch_types=[
            pltpu.VMEM((b_per_w,), jnp.int32),
            *[pltpu.VMEM((k, d), jnp.float32) for _ in range(nbuf)],
            *[pltpu.SemaphoreType.DMA for _ in range(nbuf)],
            *[pltpu.SemaphoreType.DMA for _ in range(nbuf)],
        ],
    )
    def gather_kernel(idx_hbm, table_hbm, out_hbm, idx_v, *rest):
        bufs = rest[:nbuf]
        gsems = rest[nbuf : 2 * nbuf]
        ssems = rest[2 * nbuf :]
        wid = lax.axis_index("s") * nc + lax.axis_index("c")
        base = wid * b_per_w
        pltpu.sync_copy(idx_hbm.at[pl.ds(base, b_per_w)], idx_v)
        gcp = [None] * nbuf
        scp = [None] * nbuf
        for c in range(min(nbuf, n_chunks)):
            gcp[c] = pltpu.async_copy(
                table_hbm.at[idx_v.at[pl.ds(c * k, k)]], bufs[c], gsems[c]
            )
        for c in range(n_chunks):
            s = c % nbuf
            gcp[s].wait()
            scp[s] = pltpu.async_copy(
                bufs[s], out_hbm.at[pl.ds(base + c * k, k)], ssems[s]
            )
            nx = c + nbuf
            if nx < n_chunks:
                scp[s].wait()
                gcp[s] = pltpu.async_copy(
                    table_hbm.at[idx_v.at[pl.ds(nx * k, k)]], bufs[s], gsems[s]
                )
        for c in range(max(0, n_chunks - nbuf), n_chunks):
            scp[c % nbuf].wait()

    return gather_kernel(x, table)


def _tc_combine(gathered):
    b, d = gathered.shape
    rb = 512
    grid = b // rb
    scale = math.sqrt(float(d))
    neg_log = -math.log(10000.0) / float(d)

    def body(g_ref, o_ref, p_ref, q_ref):
        blk = pl.program_id(0)
        col = lax.broadcasted_iota(jnp.int32, (1, d), 1)
        w = jnp.exp((col - (col % 2)).astype(jnp.float32) * neg_log)

        @pl.when(blk == 0)
        def _():
            r = lax.broadcasted_iota(jnp.int32, (rb, 1), 0).astype(jnp.float32)
            ang = r * w
            p_ref[...] = jnp.cos(ang)
            q_ref[...] = jnp.sin(ang)

        base_ang = (blk * rb).astype(jnp.float32) * w
        sb = jnp.sin(base_ang)
        cb = jnp.cos(base_ang)
        even = (col % 2) == 0
        a = jnp.where(even, sb, cb)
        bv = jnp.where(even, cb, -sb)
        o_ref[...] = g_ref[...] * scale + a * p_ref[...] + bv * q_ref[...]

    return pl.pallas_call(
        body,
        grid=(grid,),
        in_specs=[pl.BlockSpec((rb, d), lambda i: (i, 0))],
        out_specs=pl.BlockSpec((rb, d), lambda i: (i, 0)),
        out_shape=jax.ShapeDtypeStruct((b, d), jnp.float32),
        scratch_shapes=[
            pltpu.VMEM((rb, d), jnp.float32),
            pltpu.VMEM((rb, d), jnp.float32),
        ],
    )(gathered)


def kernel(x, table):
    x = x.astype(jnp.int32)
    gathered = _sc_gather(x, table)
    return _tc_combine(gathered)


# speedup vs baseline: 9.9722x; 1.0223x over previous
import functools
import math

import jax
import jax.numpy as jnp
from jax import lax
from jax.experimental import pallas as pl
from jax.experimental.pallas import tpu as pltpu
from jax.experimental.pallas import tpu_sc as plsc

_RB = 256
_NSLICE = 4


def _sc_gather_slice(x_slice, table):
    (b,) = x_slice.shape
    _, d = table.shape
    info = plsc.get_sparse_core_info()
    nc, ns = info.num_cores, info.num_subcores
    nw = nc * ns
    b_per_w = b // nw
    k = 16
    n_chunks = b_per_w // k
    nbuf = min(3, n_chunks)

    mesh = plsc.VectorSubcoreMesh(core_axis_name="c", subcore_axis_name="s")

    @functools.partial(
        pl.kernel,
        mesh=mesh,
        out_type=jax.ShapeDtypeStruct((b, d), jnp.float32),
        scratch_types=[
            pltpu.VMEM((b_per_w,), jnp.int32),
            *[pltpu.VMEM((k, d), jnp.float32) for _ in range(nbuf)],
            *[pltpu.SemaphoreType.DMA for _ in range(nbuf)],
            *[pltpu.SemaphoreType.DMA for _ in range(nbuf)],
        ],
    )
    def gather_kernel(idx_hbm, table_hbm, out_hbm, idx_v, *rest):
        bufs = rest[:nbuf]
        gsems = rest[nbuf : 2 * nbuf]
        ssems = rest[2 * nbuf :]
        wid = lax.axis_index("s") * nc + lax.axis_index("c")
        base = wid * b_per_w
        pltpu.sync_copy(idx_hbm.at[pl.ds(base, b_per_w)], idx_v)
        gcp = [None] * nbuf
        scp = [None] * nbuf
        for c in range(nbuf):
            gcp[c] = pltpu.async_copy(
                table_hbm.at[idx_v.at[pl.ds(c * k, k)]], bufs[c], gsems[c]
            )
        for c in range(n_chunks):
            s = c % nbuf
            gcp[s].wait()
            scp[s] = pltpu.async_copy(
                bufs[s], out_hbm.at[pl.ds(base + c * k, k)], ssems[s]
            )
            nx = c + nbuf
            if nx < n_chunks:
                scp[s].wait()
                gcp[s] = pltpu.async_copy(
                    table_hbm.at[idx_v.at[pl.ds(nx * k, k)]], bufs[s], gsems[s]
                )
        for c in range(max(0, n_chunks - nbuf), n_chunks):
            scp[c % nbuf].wait()

    return gather_kernel(x_slice, table)


def _neg_log(d):
    return -math.log(10000.0) / float(d)


def _pe_tables(d):
    nl = _neg_log(d)

    def body(p_ref, q_ref):
        col = lax.broadcasted_iota(jnp.int32, (1, d), 1)
        w = jnp.exp((col - (col % 2)).astype(jnp.float32) * nl)
        r = lax.broadcasted_iota(jnp.int32, (_RB, 1), 0).astype(jnp.float32)
        ang = r * w
        p_ref[...] = jnp.cos(ang)
        q_ref[...] = jnp.sin(ang)

    return pl.pallas_call(
        body,
        out_shape=(
            jax.ShapeDtypeStruct((_RB, d), jnp.float32),
            jax.ShapeDtypeStruct((_RB, d), jnp.float32),
        ),
    )()


def _tc_combine_slice(g, p, q, prev_out, slice_idx, b_total):
    bs, d = g.shape
    steps = bs // _RB
    blk0 = slice_idx * steps
    scale = math.sqrt(float(d))
    nl = _neg_log(d)

    def body(g_ref, p_ref, q_ref, _prev_ref, o_ref):
        i = pl.program_id(0)
        col = lax.broadcasted_iota(jnp.int32, (1, d), 1)
        w = jnp.exp((col - (col % 2)).astype(jnp.float32) * nl)
        base_ang = ((blk0 + i) * _RB).astype(jnp.float32) * w
        sb = jnp.sin(base_ang)
        cb = jnp.cos(base_ang)
        even = (col % 2) == 0
        a = jnp.where(even, sb, cb)
        bv = jnp.where(even, cb, -sb)
        o_ref[...] = g_ref[...] * scale + a * p_ref[...] + bv * q_ref[...]

    kwargs = {}
    ins = [g, p, q]
    in_specs = [
        pl.BlockSpec((_RB, d), lambda i: (i, 0)),
        pl.BlockSpec((_RB, d), lambda i: (0, 0)),
        pl.BlockSpec((_RB, d), lambda i: (0, 0)),
    ]
    if prev_out is None:
        def body0(g_ref, p_ref, q_ref, o_ref):
            return body(g_ref, p_ref, q_ref, None, o_ref)
        fn = body0
    else:
        ins.append(prev_out)
        in_specs.append(pl.BlockSpec(memory_space=pltpu.HBM))
        kwargs["input_output_aliases"] = {3: 0}
        fn = body

    return pl.pallas_call(
        fn,
        grid=(steps,),
        in_specs=in_specs,
        out_specs=pl.BlockSpec((_RB, d), lambda i: (blk0 + i, 0)),
        out_shape=jax.ShapeDtypeStruct((b_total, d), jnp.float32),
        **kwargs,
    )(*ins)


def kernel(x, table):
    (b,) = x.shape
    _, d = table.shape
    x = x.astype(jnp.int32)
    p, q = _pe_tables(d)
    bs = b // _NSLICE
    out = None
    for j in range(_NSLICE):
        xj = lax.slice(x, (j * bs,), ((j + 1) * bs,))
        gj = _sc_gather_slice(xj, table)
        out = _tc_combine_slice(gj, p, q, out, j, b)
    return out
